# baseline (speedup 1.0000x reference)
import functools

import jax
import jax.numpy as jnp
from jax import lax
from jax.experimental import pallas as pl
from jax.experimental.pallas import tpu as pltpu
from jax.experimental.pallas import tpu_sc as plsc

N = 10000
E = 320000
D = 128
H = 128
DE = 16
NG = 64

NC = 2
NS = 16
NW = NC * NS

EC = 128
EPAD = 327680
ER = EPAD // EC
RPW = ER // NW
GPC = EC // 16
EWR = EPAD // 16
WPW = EWR // NW
NPAD = 10240
RPT = NPAD // NS
CPB = 8

_mesh = plsc.VectorSubcoreMesh(core_axis_name="c", subcore_axis_name="s")


def _mlp_body(ea_ref, w1t_ref, b1_ref, w2_ref, b2_ref, ew_ref):
    h = jnp.dot(ea_ref[...], w1t_ref[...], preferred_element_type=jnp.float32)
    h = jnp.maximum(h + b1_ref[...], 0.0)
    ew_ref[...] = jnp.sum(h * w2_ref[...], axis=1, keepdims=True) + b2_ref[...]


def _c1_body(x_ref, w_ref, d0_ref, d1_ref, y_ref, dis_ref):
    deg = d0_ref[...] + d1_ref[...] + 1.0
    dis = jnp.where(deg > 0.0, lax.rsqrt(deg), 0.0)
    dis_ref[...] = dis
    y_ref[...] = jnp.dot(x_ref[...], w_ref[...],
                         preferred_element_type=jnp.float32) * dis


def _cmid_body(s0_ref, s1_ref, y_ref, dis_ref, b_ref, w_ref, out_ref):
    h = jnp.maximum(
        dis_ref[...] * (s0_ref[...] + s1_ref[...] + y_ref[...]) + b_ref[...],
        0.0)
    out_ref[...] = jnp.dot(h, w_ref[...],
                           preferred_element_type=jnp.float32) * dis_ref[...]


def _final_body(s0_ref, s1_ref, y_ref, dis_ref, b_ref, batch_ref, wr_ref,
                br_ref, out_ref):
    h = jnp.maximum(
        dis_ref[...] * (s0_ref[...] + s1_ref[...] + y_ref[...]) + b_ref[...],
        0.0)
    iota = lax.broadcasted_iota(jnp.int32, (N, NG), 1)
    oht = jnp.where(batch_ref[...] == iota, 1.0, 0.0)
    dnums = (((0,), (0,)), ((), ()))
    sums = lax.dot_general(oht, h, dnums,
                           preferred_element_type=jnp.float32)
    cnt = lax.dot_general(oht, jnp.ones((N, 1), jnp.float32), dnums,
                          preferred_element_type=jnp.float32)
    pooled = sums / jnp.maximum(cnt, 1.0)
    out_ref[...] = jnp.dot(pooled, wr_ref[...],
                           preferred_element_type=jnp.float32) + br_ref[...]


_BE = 4000
_mlp_call = pl.pallas_call(
    _mlp_body,
    grid=(E // _BE,),
    in_specs=[
        pl.BlockSpec((_BE, DE), lambda i: (i, 0)),
        pl.BlockSpec((DE, H), lambda i: (0, 0)),
        pl.BlockSpec((1, H), lambda i: (0, 0)),
        pl.BlockSpec((1, H), lambda i: (0, 0)),
        pl.BlockSpec((1, 1), lambda i: (0, 0)),
    ],
    out_specs=pl.BlockSpec((_BE, 1), lambda i: (i, 0)),
    out_shape=jax.ShapeDtypeStruct((E, 1), jnp.float32),
)

_BN = 1000
_c1_call = pl.pallas_call(
    _c1_body,
    grid=(N // _BN,),
    in_specs=[
        pl.BlockSpec((_BN, D), lambda i: (i, 0)),
        pl.BlockSpec((D, H), lambda i: (0, 0)),
        pl.BlockSpec((_BN, 1), lambda i: (i, 0)),
        pl.BlockSpec((_BN, 1), lambda i: (i, 0)),
    ],
    out_specs=(
        pl.BlockSpec((_BN, H), lambda i: (i, 0)),
        pl.BlockSpec((_BN, 1), lambda i: (i, 0)),
    ),
    out_shape=(
        jax.ShapeDtypeStruct((N, H), jnp.float32),
        jax.ShapeDtypeStruct((N, 1), jnp.float32),
    ),
)

_cmid_call = pl.pallas_call(
    _cmid_body,
    grid=(N // _BN,),
    in_specs=[
        pl.BlockSpec((_BN, H), lambda i: (i, 0)),
        pl.BlockSpec((_BN, H), lambda i: (i, 0)),
        pl.BlockSpec((_BN, H), lambda i: (i, 0)),
        pl.BlockSpec((_BN, 1), lambda i: (i, 0)),
        pl.BlockSpec((1, H), lambda i: (0, 0)),
        pl.BlockSpec((H, H), lambda i: (0, 0)),
    ],
    out_specs=pl.BlockSpec((_BN, H), lambda i: (i, 0)),
    out_shape=jax.ShapeDtypeStruct((N, H), jnp.float32),
)

_final_call = pl.pallas_call(
    _final_body,
    grid=(1,),
    in_specs=[
        pl.BlockSpec((N, H), lambda i: (0, 0)),
        pl.BlockSpec((N, H), lambda i: (0, 0)),
        pl.BlockSpec((N, H), lambda i: (0, 0)),
        pl.BlockSpec((N, 1), lambda i: (0, 0)),
        pl.BlockSpec((1, H), lambda i: (0, 0)),
        pl.BlockSpec((N, 1), lambda i: (0, 0)),
        pl.BlockSpec((H, 1), lambda i: (0, 0)),
        pl.BlockSpec((1, 1), lambda i: (0, 0)),
    ],
    out_specs=pl.BlockSpec((NG, 1), lambda i: (0, 0)),
    out_shape=jax.ShapeDtypeStruct((NG, 1), jnp.float32),
)


@functools.partial(
    pl.kernel,
    out_type=jax.ShapeDtypeStruct((NC, NPAD), jnp.float32),
    mesh=_mesh,
    scratch_types=[
        pltpu.VMEM((RPW, EC), jnp.int32),
        pltpu.VMEM((RPW, EC), jnp.float32),
        pltpu.VMEM_SHARED((NPAD,), jnp.float32),
    ],
)
def _deg_kernel(col_hbm, ew_hbm, zeros_hbm, out_hbm, colv, ewv, dacc):
    cid = lax.axis_index("c")
    sid = lax.axis_index("s")
    w = sid * NC + cid
    pltpu.sync_copy(zeros_hbm.at[pl.ds(sid * RPT, RPT)],
                    dacc.at[pl.ds(sid * RPT, RPT)])
    pltpu.sync_copy(col_hbm.at[pl.ds(w * RPW, RPW)], colv)
    pltpu.sync_copy(ew_hbm.at[pl.ds(w * RPW, RPW)], ewv)
    plsc.subcore_barrier()

    def body(j, carry):
        pltpu.sync_copy(ewv.at[j], dacc.at[colv.at[j]], add=True)
        return carry

    lax.fori_loop(0, RPW, body, 0)
    plsc.subcore_barrier()
    pltpu.sync_copy(dacc.at[pl.ds(sid * RPT, RPT)],
                    out_hbm.at[cid, pl.ds(sid * RPT, RPT)])


@functools.partial(
    pl.kernel,
    out_type=jax.ShapeDtypeStruct((NC, NPAD, H), jnp.float32),
    mesh=_mesh,
    scratch_types=[
        pltpu.VMEM((CPB, EC), jnp.int32),
        pltpu.VMEM((CPB, EC), jnp.int32),
        pltpu.VMEM((CPB * GPC, 16), jnp.float32),
        pltpu.VMEM((2, EC, H), jnp.float32),
        pltpu.VMEM_SHARED((NPAD, H), jnp.float32),
        pltpu.SemaphoreType.DMA((2,)),
    ],
)
def _msg_kernel(y_hbm, row_hbm, col_hbm, ew_hbm, zeros_hbm, out_hbm,
                rowv, colv, ewv, mbuf, acc, gsem):
    cid = lax.axis_index("c")
    sid = lax.axis_index("s")
    w = sid * NC + cid
    pltpu.sync_copy(zeros_hbm, acc.at[pl.ds(sid * RPT, RPT)])
    plsc.subcore_barrier()

    def block(b, carry):
        pltpu.sync_copy(row_hbm.at[pl.ds(w * RPW + b * CPB, CPB)], rowv)
        pltpu.sync_copy(col_hbm.at[pl.ds(w * RPW + b * CPB, CPB)], colv)
        pltpu.sync_copy(
            ew_hbm.at[pl.ds(w * WPW + b * CPB * GPC, CPB * GPC)], ewv)
        pltpu.async_copy(y_hbm.at[rowv.at[0]], mbuf.at[0], gsem.at[0])

        def chunk(j, c1):
            p = lax.rem(j, 2)
            pltpu.make_async_copy(
                y_hbm.at[rowv.at[j]], mbuf.at[p], gsem.at[p]).wait()

            @pl.when(j < CPB - 1)
            def _():
                pltpu.async_copy(y_hbm.at[rowv.at[j + 1]], mbuf.at[1 - p],
                                 gsem.at[1 - p])

            def group(g, c2):
                ev = ewv[j * GPC + g]
                for t in range(16):
                    s = ev[t]
                    k = g * 16 + t
                    for i in range(H // 16):
                        sl = pl.ds(i * 16, 16)
                        mbuf[p, k, sl] = mbuf[p, k, sl] * s
                return c2

            lax.fori_loop(0, GPC, group, 0)
            pltpu.sync_copy(mbuf.at[p], acc.at[colv.at[j]], add=True)
            return c1

        lax.fori_loop(0, CPB, chunk, 0)
        return carry

    lax.fori_loop(0, RPW // CPB, block, 0)
    plsc.subcore_barrier()
    pltpu.sync_copy(acc.at[pl.ds(sid * RPT, RPT)],
                    out_hbm.at[cid, pl.ds(sid * RPT, RPT)])


_DBG_JNP_DEG = False
_DBG_JNP_SC = False

def kernel(x, edge_index, edge_attr, batch, W1, b1, W2, b2, W3, b3,
           Wm1, bm1, Wm2, bm2, Wr, br):
    ipad = jnp.zeros((EPAD - E,), jnp.int32)
    row2d = jnp.concatenate([edge_index[0], ipad]).reshape(ER, EC)
    col2d = jnp.concatenate([edge_index[1], ipad]).reshape(ER, EC)

    ew = _mlp_call(edge_attr, Wm1.T, bm1.reshape(1, H), Wm2.reshape(1, H),
                   bm2.reshape(1, 1))
    ewp = jnp.concatenate([ew.reshape(E), jnp.zeros((EPAD - E,), jnp.float32)])
    ewd = ewp.reshape(ER, EC)
    ew2d = ewp.reshape(EWR, 16)

    zeros1 = jnp.zeros((NPAD,), jnp.float32)
    dout = _deg_kernel(col2d, ewd, zeros1)
    d0 = dout[0, :N].reshape(N, 1)
    d1 = dout[1, :N].reshape(N, 1)
    if _DBG_JNP_DEG:
        d0 = jnp.zeros((N,)).at[edge_index[1]].add(ew.reshape(E)).reshape(N, 1)
        d1 = jnp.zeros((N, 1), jnp.float32)

    y1, dis = _c1_call(x, W1.T, d0, d1)

    zeros2 = jnp.zeros((RPT, H), jnp.float32)

    def _scatter(y):
        if _DBG_JNP_SC:
            s0 = jnp.zeros((N, H)).at[edge_index[1]].add(
                y[edge_index[0]] * ew.reshape(E)[:, None])
            return jnp.stack([s0, jnp.zeros((N, H), jnp.float32)])
        s = _msg_kernel(y, row2d, col2d, ew2d, zeros2)
        return s[:, :N]

    s = _scatter(y1)
    y2 = _cmid_call(s[0], s[1], y1, dis, b1.reshape(1, H), W2.T)
    s = _scatter(y2)
    y3 = _cmid_call(s[0], s[1], y2, dis, b2.reshape(1, H), W3.T)
    s = _scatter(y3)
    out = _final_call(s[0], s[1], y3, dis, b3.reshape(1, H),
                      batch.reshape(N, 1), Wr.T, br.reshape(1, 1))
    return out.reshape(NG)


# speedup vs baseline: 1.1576x; 1.1576x over previous
import functools

import jax
import jax.numpy as jnp
from jax import lax
from jax.experimental import pallas as pl
from jax.experimental.pallas import tpu as pltpu
from jax.experimental.pallas import tpu_sc as plsc

N = 10000
E = 320000
D = 128
H = 128
DE = 16
NG = 64

NC = 2
NS = 16
NW = NC * NS

EC = 128
EPAD = 327680
ER = EPAD // EC
RPW = ER // NW
GPC = EC // 16
EWR = EPAD // 16
WPW = EWR // NW
NPAD = 10112
RPT = NPAD // NS
NPADD = 10240
RPTD = NPADD // NS
CPB = 4

_mesh = plsc.VectorSubcoreMesh(core_axis_name="c", subcore_axis_name="s")


def _mlp_body(ea_ref, w1t_ref, b1_ref, w2_ref, b2_ref, ew_ref):
    h = jnp.dot(ea_ref[...], w1t_ref[...], preferred_element_type=jnp.float32)
    h = jnp.maximum(h + b1_ref[...], 0.0)
    ew_ref[...] = jnp.sum(h * w2_ref[...], axis=1, keepdims=True) + b2_ref[...]


def _c1_body(x_ref, w_ref, d0_ref, d1_ref, y_ref, dis_ref):
    deg = d0_ref[...] + d1_ref[...] + 1.0
    dis = jnp.where(deg > 0.0, lax.rsqrt(deg), 0.0)
    dis_ref[...] = dis
    y_ref[...] = jnp.dot(x_ref[...], w_ref[...],
                         preferred_element_type=jnp.float32) * dis


def _cmid_body(s0_ref, s1_ref, y_ref, dis_ref, b_ref, w_ref, out_ref):
    h = jnp.maximum(
        dis_ref[...] * (s0_ref[...] + s1_ref[...] + y_ref[...]) + b_ref[...],
        0.0)
    out_ref[...] = jnp.dot(h, w_ref[...],
                           preferred_element_type=jnp.float32) * dis_ref[...]


def _final_body(s0_ref, s1_ref, y_ref, dis_ref, b_ref, batch_ref, wr_ref,
                br_ref, out_ref):
    h = jnp.maximum(
        dis_ref[...] * (s0_ref[...] + s1_ref[...] + y_ref[...]) + b_ref[...],
        0.0)
    iota = lax.broadcasted_iota(jnp.int32, (N, NG), 1)
    oht = jnp.where(batch_ref[...] == iota, 1.0, 0.0)
    dnums = (((0,), (0,)), ((), ()))
    sums = lax.dot_general(oht, h, dnums,
                           preferred_element_type=jnp.float32)
    cnt = lax.dot_general(oht, jnp.ones((N, 1), jnp.float32), dnums,
                          preferred_element_type=jnp.float32)
    pooled = sums / jnp.maximum(cnt, 1.0)
    out_ref[...] = jnp.dot(pooled, wr_ref[...],
                           preferred_element_type=jnp.float32) + br_ref[...]


_BE = 4000
_mlp_call = pl.pallas_call(
    _mlp_body,
    grid=(E // _BE,),
    in_specs=[
        pl.BlockSpec((_BE, DE), lambda i: (i, 0)),
        pl.BlockSpec((DE, H), lambda i: (0, 0)),
        pl.BlockSpec((1, H), lambda i: (0, 0)),
        pl.BlockSpec((1, H), lambda i: (0, 0)),
        pl.BlockSpec((1, 1), lambda i: (0, 0)),
    ],
    out_specs=pl.BlockSpec((_BE, 1), lambda i: (i, 0)),
    out_shape=jax.ShapeDtypeStruct((E, 1), jnp.float32),
)

_BN = 1000
_c1_call = pl.pallas_call(
    _c1_body,
    grid=(N // _BN,),
    in_specs=[
        pl.BlockSpec((_BN, D), lambda i: (i, 0)),
        pl.BlockSpec((D, H), lambda i: (0, 0)),
        pl.BlockSpec((_BN, 1), lambda i: (i, 0)),
        pl.BlockSpec((_BN, 1), lambda i: (i, 0)),
    ],
    out_specs=(
        pl.BlockSpec((_BN, H), lambda i: (i, 0)),
        pl.BlockSpec((_BN, 1), lambda i: (i, 0)),
    ),
    out_shape=(
        jax.ShapeDtypeStruct((N, H), jnp.float32),
        jax.ShapeDtypeStruct((N, 1), jnp.float32),
    ),
)

_cmid_call = pl.pallas_call(
    _cmid_body,
    grid=(N // _BN,),
    in_specs=[
        pl.BlockSpec((_BN, H), lambda i: (i, 0)),
        pl.BlockSpec((_BN, H), lambda i: (i, 0)),
        pl.BlockSpec((_BN, H), lambda i: (i, 0)),
        pl.BlockSpec((_BN, 1), lambda i: (i, 0)),
        pl.BlockSpec((1, H), lambda i: (0, 0)),
        pl.BlockSpec((H, H), lambda i: (0, 0)),
    ],
    out_specs=pl.BlockSpec((_BN, H), lambda i: (i, 0)),
    out_shape=jax.ShapeDtypeStruct((N, H), jnp.float32),
)

_final_call = pl.pallas_call(
    _final_body,
    grid=(1,),
    in_specs=[
        pl.BlockSpec((N, H), lambda i: (0, 0)),
        pl.BlockSpec((N, H), lambda i: (0, 0)),
        pl.BlockSpec((N, H), lambda i: (0, 0)),
        pl.BlockSpec((N, 1), lambda i: (0, 0)),
        pl.BlockSpec((1, H), lambda i: (0, 0)),
        pl.BlockSpec((N, 1), lambda i: (0, 0)),
        pl.BlockSpec((H, 1), lambda i: (0, 0)),
        pl.BlockSpec((1, 1), lambda i: (0, 0)),
    ],
    out_specs=pl.BlockSpec((NG, 1), lambda i: (0, 0)),
    out_shape=jax.ShapeDtypeStruct((NG, 1), jnp.float32),
)


@functools.partial(
    pl.kernel,
    out_type=jax.ShapeDtypeStruct((NC * NPADD,), jnp.float32),
    mesh=_mesh,
    scratch_types=[
        pltpu.VMEM((RPW, EC), jnp.int32),
        pltpu.VMEM((RPW, EC), jnp.float32),
        pltpu.VMEM_SHARED((NPADD,), jnp.float32),
    ],
)
def _deg_kernel(col_hbm, ew_hbm, zeros_hbm, out_hbm, colv, ewv, dacc):
    cid = lax.axis_index("c")
    sid = lax.axis_index("s")
    w = sid * NC + cid
    pltpu.sync_copy(zeros_hbm.at[pl.ds(sid * RPTD, RPTD)],
                    dacc.at[pl.ds(sid * RPTD, RPTD)])
    pltpu.sync_copy(col_hbm.at[pl.ds(w * RPW, RPW)], colv)
    pltpu.sync_copy(ew_hbm.at[pl.ds(w * RPW, RPW)], ewv)
    plsc.subcore_barrier()

    def body(j, carry):
        pltpu.sync_copy(ewv.at[j], dacc.at[colv.at[j]], add=True)
        return carry

    lax.fori_loop(0, RPW, body, 0)
    plsc.subcore_barrier()
    pltpu.sync_copy(dacc.at[pl.ds(sid * RPTD, RPTD)],
                    out_hbm.at[pl.ds(cid * NPADD + sid * RPTD, RPTD)])


@functools.partial(
    pl.kernel,
    out_type=jax.ShapeDtypeStruct((NC, NPAD, H), jnp.float32),
    mesh=_mesh,
    scratch_types=[
        pltpu.VMEM((2, CPB, EC), jnp.int32),
        pltpu.VMEM((2, CPB, EC), jnp.int32),
        pltpu.VMEM((CPB * GPC, 16), jnp.float32),
        pltpu.VMEM((2, EC, H), jnp.float32),
        pltpu.VMEM_SHARED((NPAD, H), jnp.float32),
        pltpu.SemaphoreType.DMA((2,)),
        pltpu.SemaphoreType.DMA((2,)),
    ],
)
def _msg_kernel(y_hbm, row_hbm, col_hbm, ew_hbm, zeros_hbm, out_hbm,
                rowv, colv, ewv, mbuf, acc, gsem, ssem):
    cid = lax.axis_index("c")
    sid = lax.axis_index("s")
    w = sid * NC + cid
    pltpu.sync_copy(zeros_hbm, acc.at[pl.ds(sid * RPT, RPT)])
    plsc.subcore_barrier()

    def load_idx(b, bp):
        pltpu.sync_copy(row_hbm.at[pl.ds(w * RPW + b * CPB, CPB)],
                        rowv.at[bp])
        pltpu.sync_copy(col_hbm.at[pl.ds(w * RPW + b * CPB, CPB)],
                        colv.at[bp])

    def load_ew(b):
        pltpu.sync_copy(
            ew_hbm.at[pl.ds(w * WPW + b * CPB * GPC, CPB * GPC)], ewv)

    load_idx(0, 0)
    load_ew(0)
    pltpu.async_copy(y_hbm.at[rowv.at[0, 0]], mbuf.at[0], gsem.at[0])

    def chunk_body(j, p):
        jb = lax.rem(j, CPB)
        bp = lax.rem(lax.div(j, CPB), 2)
        jb1 = lax.rem(j + 1, CPB)
        bp1 = lax.rem(lax.div(j + 1, CPB), 2)
        pltpu.make_async_copy(
            y_hbm.at[rowv.at[bp, jb]], mbuf.at[p], gsem.at[p]).wait()

        @pl.when(j >= 1)
        def _():
            jbm = lax.rem(j - 1, CPB)
            bpm = lax.rem(lax.div(j - 1, CPB), 2)
            pltpu.make_async_copy(mbuf.at[1 - p], acc.at[colv.at[bpm, jbm]],
                                  ssem.at[1 - p]).wait()

        @pl.when(jnp.logical_and(jb1 == 0, j + 1 < RPW))
        def _():
            load_idx(lax.div(j + 1, CPB), bp1)

        @pl.when(j + 1 < RPW)
        def _():
            pltpu.async_copy(y_hbm.at[rowv.at[bp1, jb1]], mbuf.at[1 - p],
                             gsem.at[1 - p])

        def group(g, c2):
            ev = ewv[jb * GPC + g]
            for t in range(16):
                s = ev[t]
                k = g * 16 + t
                for i in range(H // 16):
                    sl = pl.ds(i * 16, 16)
                    mbuf[p, k, sl] = mbuf[p, k, sl] * s
            return c2

        lax.fori_loop(0, GPC, group, 0)
        pltpu.async_copy(mbuf.at[p], acc.at[colv.at[bp, jb]],
                         ssem.at[p], add=True)

        @pl.when(jnp.logical_and(jb1 == 0, j + 1 < RPW))
        def _():
            load_ew(lax.div(j + 1, CPB))

    def pair(q, carry):
        chunk_body(q * 2, 0)
        chunk_body(q * 2 + 1, 1)
        return carry

    lax.fori_loop(0, RPW // 2, pair, 0)
    jj = RPW - 1
    pltpu.make_async_copy(
        mbuf.at[jj % 2], acc.at[colv.at[(jj // CPB) % 2, jj % CPB]],
        ssem.at[jj % 2]).wait()
    plsc.subcore_barrier()
    pltpu.sync_copy(acc.at[pl.ds(sid * RPT, RPT)],
                    out_hbm.at[cid, pl.ds(sid * RPT, RPT)])


_DBG_JNP_DEG = False
_DBG_JNP_SC = False

def kernel(x, edge_index, edge_attr, batch, W1, b1, W2, b2, W3, b3,
           Wm1, bm1, Wm2, bm2, Wr, br):
    ipad = jnp.zeros((EPAD - E,), jnp.int32)
    row2d = jnp.concatenate([edge_index[0], ipad]).reshape(ER, EC)
    col2d = jnp.concatenate([edge_index[1], ipad]).reshape(ER, EC)

    ew = _mlp_call(edge_attr, Wm1.T, bm1.reshape(1, H), Wm2.reshape(1, H),
                   bm2.reshape(1, 1))
    ewp = jnp.concatenate([ew.reshape(E), jnp.zeros((EPAD - E,), jnp.float32)])
    ewd = ewp.reshape(ER, EC)
    ew2d = ewp.reshape(EWR, 16)

    zeros1 = jnp.zeros((NPADD,), jnp.float32)
    dout = _deg_kernel(col2d, ewd, zeros1)
    d0 = dout[:N].reshape(N, 1)
    d1 = dout[NPADD:NPADD + N].reshape(N, 1)
    if _DBG_JNP_DEG:
        d0 = jnp.zeros((N,)).at[edge_index[1]].add(ew.reshape(E)).reshape(N, 1)
        d1 = jnp.zeros((N, 1), jnp.float32)

    y1, dis = _c1_call(x, W1.T, d0, d1)

    zeros2 = jnp.zeros((RPT, H), jnp.float32)

    def _scatter(y):
        if _DBG_JNP_SC:
            s0 = jnp.zeros((N, H)).at[edge_index[1]].add(
                y[edge_index[0]] * ew.reshape(E)[:, None])
            return jnp.stack([s0, jnp.zeros((N, H), jnp.float32)])
        s = _msg_kernel(y, row2d, col2d, ew2d, zeros2)
        return s[:, :N]

    s = _scatter(y1)
    y2 = _cmid_call(s[0], s[1], y1, dis, b1.reshape(1, H), W2.T)
    s = _scatter(y2)
    y3 = _cmid_call(s[0], s[1], y2, dis, b2.reshape(1, H), W3.T)
    s = _scatter(y3)
    out = _final_call(s[0], s[1], y3, dis, b3.reshape(1, H),
                      batch.reshape(N, 1), Wr.T, br.reshape(1, 1))
    return out.reshape(NG)
